# baseline (speedup 1.0000x reference)
import jax
import jax.numpy as jnp
from jax.experimental import pallas as pl
from jax.experimental.pallas import tpu as pltpu

B, C, H, W = 32, 5, 384, 384
LANES = 128
ROWS = H * W // LANES
SUB = ROWS // 8


def _body(t_shift, x_shift, t_mask, out, acc, cnt):
  b = pl.program_id(0)
  c = pl.program_id(1)

  @pl.when((b == 0) & (c == 0))
  def _():
    acc[...] = jnp.zeros_like(acc)
    cnt[...] = jnp.zeros_like(cnt)

  mf = jnp.where(t_mask[0] != 0.0, 1.0, 0.0)
  d = t_shift[0] - x_shift[0]
  acc[...] += (d * d * mf).reshape(SUB, 8, LANES).sum(0)

  @pl.when(c == 0)
  def _():
    cnt[...] += mf.reshape(SUB, 8, LANES).sum(0)

  @pl.when((b == B - 1) & (c == C - 2))
  def _():
    out[0] = jnp.sum(acc[...])
    out[1] = jnp.sum(cnt[...])


def kernel(inputs, target):
  t2 = target.reshape(B * C, ROWS, LANES)
  x2 = inputs.reshape(B * C, ROWS, LANES)
  plane = pl.BlockSpec((1, ROWS, LANES), lambda b, c: (C * b + 1 + c, 0, 0))
  mask = pl.BlockSpec((1, ROWS, LANES), lambda b, c: (C * b, 0, 0))
  partial = pl.pallas_call(
      _body,
      grid=(B, C - 1),
      in_specs=[plane, plane, mask],
      out_specs=pl.BlockSpec(memory_space=pltpu.SMEM),
      out_shape=jax.ShapeDtypeStruct((2,), jnp.float32),
      scratch_shapes=[
          pltpu.VMEM((8, LANES), jnp.float32),
          pltpu.VMEM((8, LANES), jnp.float32),
      ],
  )(t2, x2, t2)
  return partial[0] / (partial[1] * (C - 1))


# speedup vs baseline: 2.5186x; 2.5186x over previous
import jax
import jax.numpy as jnp
from jax.experimental import pallas as pl
from jax.experimental.pallas import tpu as pltpu

B, C, H, W = 32, 5, 384, 384


def _body(t_shift, x_shift, t_mask, out, acc, cnt):
  b = pl.program_id(0)
  c = pl.program_id(1)

  mf = jnp.where(t_mask[0, 0] != 0.0, 1.0, 0.0)
  d = t_shift[0, 0] - x_shift[0, 0]
  sq = d * d * mf

  @pl.when((b == 0) & (c == 0))
  def _():
    acc[...] = sq
    cnt[...] = mf

  @pl.when((b > 0) | (c > 0))
  def _():
    acc[...] += sq

    @pl.when(c == 0)
    def _():
      cnt[...] += mf

  @pl.when((b == B - 1) & (c == C - 2))
  def _():
    out[0] = jnp.sum(acc[...])
    out[1] = jnp.sum(cnt[...])


def kernel(inputs, target):
  plane = pl.BlockSpec((1, 1, H, W), lambda b, c: (b, 1 + c, 0, 0))
  mask = pl.BlockSpec((1, 1, H, W), lambda b, c: (b, 0, 0, 0))
  partial = pl.pallas_call(
      _body,
      grid=(B, C - 1),
      in_specs=[plane, plane, mask],
      out_specs=pl.BlockSpec(memory_space=pltpu.SMEM),
      out_shape=jax.ShapeDtypeStruct((2,), jnp.float32),
      scratch_shapes=[
          pltpu.VMEM((H, W), jnp.float32),
          pltpu.VMEM((H, W), jnp.float32),
      ],
  )(target, inputs, target)
  return partial[0] / (partial[1] * (C - 1))


# speedup vs baseline: 4.9493x; 1.9651x over previous
import jax
import jax.numpy as jnp
from jax import lax
from jax.experimental import pallas as pl
from jax.experimental.pallas import tpu as pltpu

B, C, H, W = 32, 5, 384, 384


def _x_copy(x_hbm, xbuf, sem, b, slot):
  return pltpu.make_async_copy(
      x_hbm.at[b, pl.ds(1, C - 1)], xbuf.at[slot], sem.at[slot])


def _body(t_ref, x_hbm, out, xbuf, acc, cnt, sem):
  b = pl.program_id(0)
  slot = lax.rem(b, 2)
  nxt = 1 - slot

  @pl.when(b == 0)
  def _():
    _x_copy(x_hbm, xbuf, sem, 0, 0).start()

  @pl.when(b + 1 < B)
  def _():
    _x_copy(x_hbm, xbuf, sem, b + 1, nxt).start()

  _x_copy(x_hbm, xbuf, sem, b, slot).wait()

  mf = jnp.where(t_ref[0, 0] != 0.0, 1.0, 0.0)
  s = None
  for c in range(C - 1):
    d = t_ref[0, 1 + c] - xbuf[slot, c]
    s = d * d if s is None else s + d * d
  sq = s * mf

  @pl.when(b == 0)
  def _():
    acc[...] = sq
    cnt[...] = mf

  @pl.when(b > 0)
  def _():
    acc[...] += sq
    cnt[...] += mf

  @pl.when(b == B - 1)
  def _():
    out[0] = jnp.sum(acc[...])
    out[1] = jnp.sum(cnt[...])


def kernel(inputs, target):
  partial = pl.pallas_call(
      _body,
      grid=(B,),
      in_specs=[
          pl.BlockSpec((1, C, H, W), lambda b: (b, 0, 0, 0)),
          pl.BlockSpec(memory_space=pl.ANY),
      ],
      out_specs=pl.BlockSpec(memory_space=pltpu.SMEM),
      out_shape=jax.ShapeDtypeStruct((2,), jnp.float32),
      scratch_shapes=[
          pltpu.VMEM((2, C - 1, H, W), jnp.float32),
          pltpu.VMEM((H, W), jnp.float32),
          pltpu.VMEM((H, W), jnp.float32),
          pltpu.SemaphoreType.DMA((2,)),
      ],
  )(target, inputs)
  return partial[0] / (partial[1] * (C - 1))
